# baseline (speedup 1.0000x reference)
import functools

import jax
import jax.numpy as jnp
from jax import lax
from jax.experimental import pallas as pl
from jax.experimental.pallas import tpu as pltpu
from jax.experimental.pallas import tpu_sc as plsc

N = 10000
D = 128
E = 160000
NC = 2
NS = 16
EPT = E // NS
K = 80
NCHUNK = EPT // K
RPT = 624
ZR = 208

_mesh = plsc.VectorSubcoreMesh(core_axis_name="c", subcore_axis_name="s")


def _deg_body(e0_hbm, e1_hbm, deg_hbm, ones_v, id0, id1, id2, id3,
              dsem0, dsem1, dsem2, dsem3, ssem0, ssem1, ssem2, ssem3, acc_sh):
    c = lax.axis_index("c")
    s = lax.axis_index("s")
    idbuf = (id0, id1, id2, id3)
    dsem = (dsem0, dsem1, dsem2, dsem3)
    ssem = (ssem0, ssem1, ssem2, ssem3)

    ebase = E + s * EPT

    def start_idst(j, slot):
        @pl.when(c == 0)
        def _():
            pltpu.async_copy(e0_hbm.at[pl.ds(ebase + j * K, K)], idbuf[slot],
                             dsem[slot])

        @pl.when(c == 1)
        def _():
            pltpu.async_copy(e1_hbm.at[pl.ds(ebase + j * K, K)], idbuf[slot],
                             dsem[slot])

    def wait_idst(slot):
        pltpu.make_async_copy(e0_hbm.at[pl.ds(0, K)], idbuf[slot],
                              dsem[slot]).wait()

    def start_scatter(slot):
        pltpu.async_copy(ones_v.at[pl.ds(0, K)], acc_sh.at[idbuf[slot]],
                         ssem[slot], add=True)

    def wait_scatter(slot):
        pltpu.make_async_copy(e0_hbm.at[pl.ds(0, K)],
                              idbuf[slot].at[pl.ds(0, K)],
                              ssem[slot]).wait()

    start_idst(0, 0)
    start_idst(1, 1)

    def fill(i, _):
        ones_v[pl.ds(i * 16, 16)] = jnp.full((16,), 1.0, jnp.float32)
        return 0
    lax.fori_loop(0, 40, fill, 0)

    pltpu.sync_copy(ones_v.at[pl.ds(0, 624)], acc_sh.at[pl.ds(s * 624, 624)])

    @pl.when(s == NS - 1)
    def _():
        pltpu.sync_copy(ones_v.at[pl.ds(0, 16)], acc_sh.at[pl.ds(9984, 16)])

    plsc.subcore_barrier()

    def step(j, b, drain, pf):
        wait_idst(b)
        start_scatter(b)
        if drain:
            wait_scatter((b + 2) % 4)
        if pf:
            start_idst(j + 2, (b + 2) % 4)

    step(0, 0, False, True)
    step(1, 1, False, True)

    def steady(i, _):
        for b4 in range(4):
            j = 2 + i * 4 + b4
            step(j, (2 + b4) % 4, True, True)
        return 0
    lax.fori_loop(0, 30, steady, 0)

    step(NCHUNK - 3, (NCHUNK - 3) % 4, True, True)
    step(NCHUNK - 2, (NCHUNK - 2) % 4, True, False)
    step(NCHUNK - 1, (NCHUNK - 1) % 4, True, False)
    wait_scatter((NCHUNK - 2) % 4)
    wait_scatter((NCHUNK - 1) % 4)

    plsc.subcore_barrier()

    obase = c * N
    pltpu.sync_copy(acc_sh.at[pl.ds(s * 624, 624)], ones_v.at[pl.ds(0, 624)])
    pltpu.sync_copy(ones_v.at[pl.ds(0, 624)],
                    deg_hbm.at[pl.ds(obase + s * 624, 624)])

    @pl.when(s == NS - 1)
    def _():
        pltpu.sync_copy(acc_sh.at[pl.ds(9984, 16)], ones_v.at[pl.ds(624, 16)])
        pltpu.sync_copy(ones_v.at[pl.ds(624, 16)],
                        deg_hbm.at[pl.ds(obase + 9984, 16)])


_sc_degree = functools.partial(
    pl.kernel,
    out_type=jax.ShapeDtypeStruct((NC * N,), jnp.float32),
    mesh=_mesh,
    scratch_types=(
        [pltpu.VMEM((640,), jnp.float32)]
        + [pltpu.VMEM((K,), jnp.int32)] * 4
        + [pltpu.SemaphoreType.DMA] * 8
        + [pltpu.VMEM_SHARED((N,), jnp.float32)]
    ),
)(_deg_body)


def _agg_body(y0_hbm, y1_hbm, e0_hbm, e1_hbm, agg_hbm,
              is0, is1, is2, is3, id0, id1, id2, id3,
              r0, r1, r2, r3, acc_sh,
              isem0, isem1, isem2, isem3, dsem0, dsem1, dsem2, dsem3,
              gsem0, gsem1, gsem2, gsem3, ssem0, ssem1, ssem2, ssem3):
    c = lax.axis_index("c")
    s = lax.axis_index("s")
    rows = (r0, r1, r2, r3)
    isbuf = (is0, is1, is2, is3)
    idbuf = (id0, id1, id2, id3)
    isem = (isem0, isem1, isem2, isem3)
    dsem = (dsem0, dsem1, dsem2, dsem3)
    gsem = (gsem0, gsem1, gsem2, gsem3)
    ssem = (ssem0, ssem1, ssem2, ssem3)

    sbase = s * EPT
    dbase = E + s * EPT

    def start_isrc(j, slot):
        @pl.when(c == 0)
        def _():
            pltpu.async_copy(e0_hbm.at[pl.ds(sbase + j * K, K)], isbuf[slot],
                             isem[slot])

        @pl.when(c == 1)
        def _():
            pltpu.async_copy(e1_hbm.at[pl.ds(sbase + j * K, K)], isbuf[slot],
                             isem[slot])

    def wait_isrc(slot):
        pltpu.make_async_copy(e0_hbm.at[pl.ds(0, K)], isbuf[slot],
                              isem[slot]).wait()

    def start_idst(j, slot):
        @pl.when(c == 0)
        def _():
            pltpu.async_copy(e0_hbm.at[pl.ds(dbase + j * K, K)], idbuf[slot],
                             dsem[slot])

        @pl.when(c == 1)
        def _():
            pltpu.async_copy(e1_hbm.at[pl.ds(dbase + j * K, K)], idbuf[slot],
                             dsem[slot])

    def wait_idst(slot):
        pltpu.make_async_copy(e0_hbm.at[pl.ds(0, K)], idbuf[slot],
                              dsem[slot]).wait()

    def start_gather(slot):
        @pl.when(c == 0)
        def _():
            pltpu.async_copy(y0_hbm.at[isbuf[slot]], rows[slot], gsem[slot])

        @pl.when(c == 1)
        def _():
            pltpu.async_copy(y1_hbm.at[isbuf[slot]], rows[slot], gsem[slot])

    def wait_gather(slot):
        pltpu.make_async_copy(y0_hbm.at[pl.ds(0, K)], rows[slot],
                              gsem[slot]).wait()

    def start_scatter(slot):
        pltpu.async_copy(rows[slot], acc_sh.at[idbuf[slot]], ssem[slot],
                         add=True)

    def wait_scatter(slot):
        pltpu.make_async_copy(y0_hbm.at[pl.ds(0, K)], rows[slot],
                              ssem[slot]).wait()

    start_isrc(0, 0)
    start_isrc(1, 1)
    start_isrc(2, 2)
    start_idst(0, 0)
    start_idst(1, 1)

    def zfill(i, _):
        r = i // 8
        col = (i % 8) * 16
        r0[r, pl.ds(col, 16)] = jnp.zeros((16,), jnp.float32)
        return 0
    lax.fori_loop(0, K * 8, zfill, 0)

    rbase = s * RPT

    def zcopy(k, _):
        pltpu.sync_copy(r0, acc_sh.at[pl.ds(rbase + k * K, K)])
        return 0
    lax.fori_loop(0, 7, zcopy, 0)
    pltpu.sync_copy(r0.at[pl.ds(0, 64)], acc_sh.at[pl.ds(rbase + 560, 64)])

    @pl.when(s == NS - 1)
    def _():
        pltpu.sync_copy(r0.at[pl.ds(0, 16)], acc_sh.at[pl.ds(9984, 16)])

    wait_isrc(0)
    start_gather(0)
    wait_isrc(1)
    start_gather(1)

    plsc.subcore_barrier()

    def step(j, b, drain, pf_id, pf_g, pf_is):
        wait_gather(b)
        wait_idst(b)
        start_scatter(b)
        if drain:
            wait_scatter((b + 2) % 4)
        if pf_id:
            start_idst(j + 2, (b + 2) % 4)
        if pf_g:
            wait_isrc((b + 2) % 4)
            start_gather((b + 2) % 4)
        if pf_is:
            start_isrc(j + 3, (b + 3) % 4)

    step(0, 0, False, True, True, True)
    step(1, 1, False, True, True, True)

    def steady(i, _):
        for b4 in range(4):
            j = 2 + i * 4 + b4
            step(j, (2 + b4) % 4, True, True, True, True)
        return 0
    lax.fori_loop(0, 30, steady, 0)

    step(NCHUNK - 3, (NCHUNK - 3) % 4, True, True, True, False)
    step(NCHUNK - 2, (NCHUNK - 2) % 4, True, False, False, False)
    step(NCHUNK - 1, (NCHUNK - 1) % 4, True, False, False, False)
    wait_scatter((NCHUNK - 2) % 4)
    wait_scatter((NCHUNK - 1) % 4)

    plsc.subcore_barrier()

    obase = c * N + rbase

    def ocopy(k, _):
        pltpu.sync_copy(acc_sh.at[pl.ds(rbase + k * K, K)], r0)
        pltpu.sync_copy(r0, agg_hbm.at[pl.ds(obase + k * K, K)])
        return 0
    lax.fori_loop(0, 7, ocopy, 0)
    pltpu.sync_copy(acc_sh.at[pl.ds(rbase + 560, 64)], r1.at[pl.ds(0, 64)])
    pltpu.sync_copy(r1.at[pl.ds(0, 64)], agg_hbm.at[pl.ds(obase + 560, 64)])

    @pl.when(s == NS - 1)
    def _():
        pltpu.sync_copy(acc_sh.at[pl.ds(9984, 16)], r2.at[pl.ds(0, 16)])
        pltpu.sync_copy(r2.at[pl.ds(0, 16)],
                        agg_hbm.at[pl.ds(c * N + 9984, 16)])


_sc_agg = functools.partial(
    pl.kernel,
    out_type=jax.ShapeDtypeStruct((NC * N, D), jnp.float32),
    mesh=_mesh,
    scratch_types=(
        [pltpu.VMEM((K,), jnp.int32)] * 8
        + [pltpu.VMEM((K, D), jnp.float32)] * 4
        + [pltpu.VMEM_SHARED((N, D), jnp.float32)]
        + [pltpu.SemaphoreType.DMA] * 16
    ),
)(_agg_body)


RB = 2000
GRID = N // RB

_full = pl.BlockSpec((D, D), lambda i: (0, 0))
_bias = pl.BlockSpec((1, D), lambda i: (0, 0))
_rows = pl.BlockSpec((RB, D), lambda i: (i, 0))
_rows2 = pl.BlockSpec((NC, RB, D), lambda i: (0, i, 0))
_deg0 = pl.BlockSpec((RB, 1), lambda i: (i, 0))
_deg1 = pl.BlockSpec((RB, 1), lambda i: (i + GRID, 0))


def _prep_a_body(x_ref, wemb_ref, bemb_ref, w00_ref, w01_ref, xw_ref):
    h = jnp.dot(x_ref[...], wemb_ref[...], preferred_element_type=jnp.float32)
    h = jnp.maximum(h + bemb_ref[...], 0.0)
    xw_ref[0] = jnp.dot(h, w00_ref[...], preferred_element_type=jnp.float32)
    xw_ref[1] = jnp.dot(h, w01_ref[...], preferred_element_type=jnp.float32)


_tc_prep_a = pl.pallas_call(
    _prep_a_body,
    grid=(GRID,),
    in_specs=[_rows, _full, _bias, _full, _full],
    out_specs=_rows2,
    out_shape=jax.ShapeDtypeStruct((NC, N, D), jnp.float32),
)


def _prep_b_body(xw_ref, b00_ref, b01_ref, deg0_ref, deg1_ref,
                 y0_ref, y1_ref, selfsum_ref):
    xw0 = xw_ref[0]
    xw1 = xw_ref[1]
    d0 = lax.rsqrt(deg0_ref[...])
    d1 = lax.rsqrt(deg1_ref[...])
    y0_ref[...] = xw0 * d0
    y1_ref[...] = xw1 * d1
    selfsum_ref[...] = (xw0 * (d0 * d0) + xw1 * (d1 * d1)
                        + b00_ref[...] + b01_ref[...])


_tc_prep_b = pl.pallas_call(
    _prep_b_body,
    grid=(GRID,),
    in_specs=[_rows2, _bias, _bias, _deg0, _deg1],
    out_specs=[_rows, _rows, _rows],
    out_shape=[jax.ShapeDtypeStruct((N, D), jnp.float32)] * 3,
)


def _mid_body(agg_ref, selfsum_ref, deg0_ref, deg1_ref,
              w10_ref, b10_ref, w11_ref, b11_ref,
              y0_ref, y1_ref, selfsum2_ref):
    d0 = lax.rsqrt(deg0_ref[...])
    d1 = lax.rsqrt(deg1_ref[...])
    h1 = jnp.maximum(agg_ref[0] * d0 + agg_ref[1] * d1 + selfsum_ref[...], 0.0)
    xw0 = jnp.dot(h1, w10_ref[...], preferred_element_type=jnp.float32)
    xw1 = jnp.dot(h1, w11_ref[...], preferred_element_type=jnp.float32)
    y0_ref[...] = xw0 * d0
    y1_ref[...] = xw1 * d1
    selfsum2_ref[...] = (xw0 * (d0 * d0) + xw1 * (d1 * d1)
                         + b10_ref[...] + b11_ref[...])


_tc_mid = pl.pallas_call(
    _mid_body,
    grid=(GRID,),
    in_specs=[_rows2, _rows, _deg0, _deg1, _full, _bias, _full, _bias],
    out_specs=[_rows, _rows, _rows],
    out_shape=[jax.ShapeDtypeStruct((N, D), jnp.float32)] * 3,
)


def _final_body(agg_ref, selfsum_ref, deg0_ref, deg1_ref,
                wlin_ref, blin_ref, out_ref):
    d0 = lax.rsqrt(deg0_ref[...])
    d1 = lax.rsqrt(deg1_ref[...])
    h2 = jnp.maximum(agg_ref[0] * d0 + agg_ref[1] * d1 + selfsum_ref[...], 0.0)
    out_ref[...] = (jnp.dot(h2, wlin_ref[...], preferred_element_type=jnp.float32)
                    + blin_ref[...])


_tc_final = pl.pallas_call(
    _final_body,
    grid=(GRID,),
    in_specs=[_rows2, _rows, _deg0, _deg1, _full, _bias],
    out_specs=_rows,
    out_shape=jax.ShapeDtypeStruct((N, D), jnp.float32),
)


def kernel(x, edge_index_r0, edge_index_r1, W_emb, b_emb, W00, b00, W01, b01,
           W10, b10, W11, b11, W_lin, b_lin):
    e0f = edge_index_r0.reshape(2 * E)
    e1f = edge_index_r1.reshape(2 * E)

    deg = _sc_degree(e0f, e1f).reshape(NC * N, 1)

    b_emb2 = b_emb[None, :]
    b00_2, b01_2 = b00[None, :], b01[None, :]
    b10_2, b11_2 = b10[None, :], b11[None, :]
    b_lin2 = b_lin[None, :]

    xw = _tc_prep_a(x, W_emb, b_emb2, W00, W01)
    y0, y1, selfsum = _tc_prep_b(xw, b00_2, b01_2, deg, deg)
    agg = _sc_agg(y0, y1, e0f, e1f)

    y0b, y1b, selfsum2 = _tc_mid(agg.reshape(NC, N, D), selfsum, deg, deg,
                                 W10, b10_2, W11, b11_2)
    agg2 = _sc_agg(y0b, y1b, e0f, e1f)

    return _tc_final(agg2.reshape(NC, N, D), selfsum2, deg, deg,
                     W_lin, b_lin2)


# speedup vs baseline: 1.0817x; 1.0817x over previous
import functools

import jax
import jax.numpy as jnp
from jax import lax
from jax.experimental import pallas as pl
from jax.experimental.pallas import tpu as pltpu
from jax.experimental.pallas import tpu_sc as plsc

N = 10000
D = 128
E = 160000
NC = 2
NS = 16
EPT = E // NS
K = 80
NCHUNK = EPT // K
RPT = 624
ZR = 208

_mesh = plsc.VectorSubcoreMesh(core_axis_name="c", subcore_axis_name="s")


def _deg_body(e0_hbm, e1_hbm, deg_hbm, ones_v, idx_all, lsem, ssem, acc_sh):
    c = lax.axis_index("c")
    s = lax.axis_index("s")

    ebase = E + s * EPT

    @pl.when(c == 0)
    def _():
        pltpu.async_copy(e0_hbm.at[pl.ds(ebase, EPT)], idx_all, lsem)

    @pl.when(c == 1)
    def _():
        pltpu.async_copy(e1_hbm.at[pl.ds(ebase, EPT)], idx_all, lsem)

    def fill(i, _):
        ones_v[pl.ds(i * 16, 16)] = jnp.full((16,), 1.0, jnp.float32)
        return 0
    lax.fori_loop(0, 40, fill, 0)

    pltpu.sync_copy(ones_v.at[pl.ds(0, 624)], acc_sh.at[pl.ds(s * 624, 624)])

    @pl.when(s == NS - 1)
    def _():
        pltpu.sync_copy(ones_v.at[pl.ds(0, 16)], acc_sh.at[pl.ds(9984, 16)])

    pltpu.make_async_copy(e0_hbm.at[pl.ds(0, EPT)], idx_all, lsem).wait()

    plsc.subcore_barrier()

    def drain25():
        pltpu.make_async_copy(e0_hbm.at[pl.ds(0, 400)],
                              idx_all.at[pl.ds(0, 400)], ssem).wait()

    def outer(i, _):
        for u in range(25):
            vec = idx_all[pl.ds((i * 25 + u) * 16, 16)]
            pltpu.async_copy(ones_v.at[pl.ds(0, 16)], acc_sh.at[vec],
                             ssem, add=True)

        @pl.when(i > 0)
        def _():
            drain25()
        return 0
    lax.fori_loop(0, EPT // (25 * 16), outer, 0)
    drain25()

    plsc.subcore_barrier()

    obase = c * N
    pltpu.sync_copy(acc_sh.at[pl.ds(s * 624, 624)], ones_v.at[pl.ds(0, 624)])
    pltpu.sync_copy(ones_v.at[pl.ds(0, 624)],
                    deg_hbm.at[pl.ds(obase + s * 624, 624)])

    @pl.when(s == NS - 1)
    def _():
        pltpu.sync_copy(acc_sh.at[pl.ds(9984, 16)], ones_v.at[pl.ds(624, 16)])
        pltpu.sync_copy(ones_v.at[pl.ds(624, 16)],
                        deg_hbm.at[pl.ds(obase + 9984, 16)])


_sc_degree = functools.partial(
    pl.kernel,
    out_type=jax.ShapeDtypeStruct((NC * N,), jnp.float32),
    mesh=_mesh,
    scratch_types=(
        [pltpu.VMEM((640,), jnp.float32),
         pltpu.VMEM((EPT,), jnp.int32)]
        + [pltpu.SemaphoreType.DMA] * 2
        + [pltpu.VMEM_SHARED((N,), jnp.float32)]
    ),
)(_deg_body)


def _agg_body(y0_hbm, y1_hbm, e0_hbm, e1_hbm, agg_hbm,
              is0, is1, is2, is3, id0, id1, id2, id3,
              r0, r1, r2, r3, acc_sh,
              isem0, isem1, isem2, isem3, dsem0, dsem1, dsem2, dsem3,
              gsem0, gsem1, gsem2, gsem3, ssem0, ssem1, ssem2, ssem3):
    c = lax.axis_index("c")
    s = lax.axis_index("s")
    rows = (r0, r1, r2, r3)
    isbuf = (is0, is1, is2, is3)
    idbuf = (id0, id1, id2, id3)
    isem = (isem0, isem1, isem2, isem3)
    dsem = (dsem0, dsem1, dsem2, dsem3)
    gsem = (gsem0, gsem1, gsem2, gsem3)
    ssem = (ssem0, ssem1, ssem2, ssem3)

    sbase = s * EPT
    dbase = E + s * EPT

    def start_isrc(j, slot):
        @pl.when(c == 0)
        def _():
            pltpu.async_copy(e0_hbm.at[pl.ds(sbase + j * K, K)], isbuf[slot],
                             isem[slot])

        @pl.when(c == 1)
        def _():
            pltpu.async_copy(e1_hbm.at[pl.ds(sbase + j * K, K)], isbuf[slot],
                             isem[slot])

    def wait_isrc(slot):
        pltpu.make_async_copy(e0_hbm.at[pl.ds(0, K)], isbuf[slot],
                              isem[slot]).wait()

    def start_idst(j, slot):
        @pl.when(c == 0)
        def _():
            pltpu.async_copy(e0_hbm.at[pl.ds(dbase + j * K, K)], idbuf[slot],
                             dsem[slot])

        @pl.when(c == 1)
        def _():
            pltpu.async_copy(e1_hbm.at[pl.ds(dbase + j * K, K)], idbuf[slot],
                             dsem[slot])

    def wait_idst(slot):
        pltpu.make_async_copy(e0_hbm.at[pl.ds(0, K)], idbuf[slot],
                              dsem[slot]).wait()

    def start_gather(slot):
        @pl.when(c == 0)
        def _():
            pltpu.async_copy(y0_hbm.at[isbuf[slot]], rows[slot], gsem[slot])

        @pl.when(c == 1)
        def _():
            pltpu.async_copy(y1_hbm.at[isbuf[slot]], rows[slot], gsem[slot])

    def wait_gather(slot):
        pltpu.make_async_copy(y0_hbm.at[pl.ds(0, K)], rows[slot],
                              gsem[slot]).wait()

    def start_scatter(slot):
        pltpu.async_copy(rows[slot], acc_sh.at[idbuf[slot]], ssem[slot],
                         add=True)

    def wait_scatter(slot):
        pltpu.make_async_copy(y0_hbm.at[pl.ds(0, K)], rows[slot],
                              ssem[slot]).wait()

    start_isrc(0, 0)
    start_isrc(1, 1)
    start_isrc(2, 2)
    start_idst(0, 0)
    start_idst(1, 1)

    def zfill(i, _):
        r = i // 8
        col = (i % 8) * 16
        r0[r, pl.ds(col, 16)] = jnp.zeros((16,), jnp.float32)
        return 0
    lax.fori_loop(0, K * 8, zfill, 0)

    rbase = s * RPT

    def zcopy(k, _):
        pltpu.sync_copy(r0, acc_sh.at[pl.ds(rbase + k * K, K)])
        return 0
    lax.fori_loop(0, 7, zcopy, 0)
    pltpu.sync_copy(r0.at[pl.ds(0, 64)], acc_sh.at[pl.ds(rbase + 560, 64)])

    @pl.when(s == NS - 1)
    def _():
        pltpu.sync_copy(r0.at[pl.ds(0, 16)], acc_sh.at[pl.ds(9984, 16)])

    wait_isrc(0)
    start_gather(0)
    wait_isrc(1)
    start_gather(1)

    plsc.subcore_barrier()

    def step(j, b, drain, pf_id, pf_g, pf_is):
        wait_gather(b)
        wait_idst(b)
        start_scatter(b)
        if drain:
            wait_scatter((b + 2) % 4)
        if pf_id:
            start_idst(j + 2, (b + 2) % 4)
        if pf_g:
            wait_isrc((b + 2) % 4)
            start_gather((b + 2) % 4)
        if pf_is:
            start_isrc(j + 3, (b + 3) % 4)

    step(0, 0, False, True, True, True)
    step(1, 1, False, True, True, True)

    def steady(i, _):
        for b4 in range(4):
            j = 2 + i * 4 + b4
            step(j, (2 + b4) % 4, True, True, True, True)
        return 0
    lax.fori_loop(0, 30, steady, 0)

    step(NCHUNK - 3, (NCHUNK - 3) % 4, True, True, True, False)
    step(NCHUNK - 2, (NCHUNK - 2) % 4, True, False, False, False)
    step(NCHUNK - 1, (NCHUNK - 1) % 4, True, False, False, False)
    wait_scatter((NCHUNK - 2) % 4)
    wait_scatter((NCHUNK - 1) % 4)

    plsc.subcore_barrier()

    obase = c * N + rbase

    def ocopy(k, _):
        pltpu.sync_copy(acc_sh.at[pl.ds(rbase + k * K, K)], r0)
        pltpu.sync_copy(r0, agg_hbm.at[pl.ds(obase + k * K, K)])
        return 0
    lax.fori_loop(0, 7, ocopy, 0)
    pltpu.sync_copy(acc_sh.at[pl.ds(rbase + 560, 64)], r1.at[pl.ds(0, 64)])
    pltpu.sync_copy(r1.at[pl.ds(0, 64)], agg_hbm.at[pl.ds(obase + 560, 64)])

    @pl.when(s == NS - 1)
    def _():
        pltpu.sync_copy(acc_sh.at[pl.ds(9984, 16)], r2.at[pl.ds(0, 16)])
        pltpu.sync_copy(r2.at[pl.ds(0, 16)],
                        agg_hbm.at[pl.ds(c * N + 9984, 16)])


_sc_agg = functools.partial(
    pl.kernel,
    out_type=jax.ShapeDtypeStruct((NC * N, D), jnp.float32),
    mesh=_mesh,
    scratch_types=(
        [pltpu.VMEM((K,), jnp.int32)] * 8
        + [pltpu.VMEM((K, D), jnp.float32)] * 4
        + [pltpu.VMEM_SHARED((N, D), jnp.float32)]
        + [pltpu.SemaphoreType.DMA] * 16
    ),
)(_agg_body)


RB = 2000
GRID = N // RB

_full = pl.BlockSpec((D, D), lambda i: (0, 0))
_bias = pl.BlockSpec((1, D), lambda i: (0, 0))
_rows = pl.BlockSpec((RB, D), lambda i: (i, 0))
_rows2 = pl.BlockSpec((NC, RB, D), lambda i: (0, i, 0))
_deg0 = pl.BlockSpec((RB, 1), lambda i: (i, 0))
_deg1 = pl.BlockSpec((RB, 1), lambda i: (i + GRID, 0))


def _prep_a_body(x_ref, wemb_ref, bemb_ref, w00_ref, w01_ref, xw_ref):
    h = jnp.dot(x_ref[...], wemb_ref[...], preferred_element_type=jnp.float32)
    h = jnp.maximum(h + bemb_ref[...], 0.0)
    xw_ref[0] = jnp.dot(h, w00_ref[...], preferred_element_type=jnp.float32)
    xw_ref[1] = jnp.dot(h, w01_ref[...], preferred_element_type=jnp.float32)


_tc_prep_a = pl.pallas_call(
    _prep_a_body,
    grid=(GRID,),
    in_specs=[_rows, _full, _bias, _full, _full],
    out_specs=_rows2,
    out_shape=jax.ShapeDtypeStruct((NC, N, D), jnp.float32),
)


def _prep_b_body(xw_ref, b00_ref, b01_ref, deg0_ref, deg1_ref,
                 y0_ref, y1_ref, selfsum_ref):
    xw0 = xw_ref[0]
    xw1 = xw_ref[1]
    d0 = lax.rsqrt(deg0_ref[...])
    d1 = lax.rsqrt(deg1_ref[...])
    y0_ref[...] = xw0 * d0
    y1_ref[...] = xw1 * d1
    selfsum_ref[...] = (xw0 * (d0 * d0) + xw1 * (d1 * d1)
                        + b00_ref[...] + b01_ref[...])


_tc_prep_b = pl.pallas_call(
    _prep_b_body,
    grid=(GRID,),
    in_specs=[_rows2, _bias, _bias, _deg0, _deg1],
    out_specs=[_rows, _rows, _rows],
    out_shape=[jax.ShapeDtypeStruct((N, D), jnp.float32)] * 3,
)


def _mid_body(agg_ref, selfsum_ref, deg0_ref, deg1_ref,
              w10_ref, b10_ref, w11_ref, b11_ref,
              y0_ref, y1_ref, selfsum2_ref):
    d0 = lax.rsqrt(deg0_ref[...])
    d1 = lax.rsqrt(deg1_ref[...])
    h1 = jnp.maximum(agg_ref[0] * d0 + agg_ref[1] * d1 + selfsum_ref[...], 0.0)
    xw0 = jnp.dot(h1, w10_ref[...], preferred_element_type=jnp.float32)
    xw1 = jnp.dot(h1, w11_ref[...], preferred_element_type=jnp.float32)
    y0_ref[...] = xw0 * d0
    y1_ref[...] = xw1 * d1
    selfsum2_ref[...] = (xw0 * (d0 * d0) + xw1 * (d1 * d1)
                         + b10_ref[...] + b11_ref[...])


_tc_mid = pl.pallas_call(
    _mid_body,
    grid=(GRID,),
    in_specs=[_rows2, _rows, _deg0, _deg1, _full, _bias, _full, _bias],
    out_specs=[_rows, _rows, _rows],
    out_shape=[jax.ShapeDtypeStruct((N, D), jnp.float32)] * 3,
)


def _final_body(agg_ref, selfsum_ref, deg0_ref, deg1_ref,
                wlin_ref, blin_ref, out_ref):
    d0 = lax.rsqrt(deg0_ref[...])
    d1 = lax.rsqrt(deg1_ref[...])
    h2 = jnp.maximum(agg_ref[0] * d0 + agg_ref[1] * d1 + selfsum_ref[...], 0.0)
    out_ref[...] = (jnp.dot(h2, wlin_ref[...], preferred_element_type=jnp.float32)
                    + blin_ref[...])


_tc_final = pl.pallas_call(
    _final_body,
    grid=(GRID,),
    in_specs=[_rows2, _rows, _deg0, _deg1, _full, _bias],
    out_specs=_rows,
    out_shape=jax.ShapeDtypeStruct((N, D), jnp.float32),
)


def kernel(x, edge_index_r0, edge_index_r1, W_emb, b_emb, W00, b00, W01, b01,
           W10, b10, W11, b11, W_lin, b_lin):
    e0f = edge_index_r0.reshape(2 * E)
    e1f = edge_index_r1.reshape(2 * E)

    deg = _sc_degree(e0f, e1f).reshape(NC * N, 1)

    b_emb2 = b_emb[None, :]
    b00_2, b01_2 = b00[None, :], b01[None, :]
    b10_2, b11_2 = b10[None, :], b11[None, :]
    b_lin2 = b_lin[None, :]

    xw = _tc_prep_a(x, W_emb, b_emb2, W00, W01)
    y0, y1, selfsum = _tc_prep_b(xw, b00_2, b01_2, deg, deg)
    agg = _sc_agg(y0, y1, e0f, e1f)

    y0b, y1b, selfsum2 = _tc_mid(agg.reshape(NC, N, D), selfsum, deg, deg,
                                 W10, b10_2, W11, b11_2)
    agg2 = _sc_agg(y0b, y1b, e0f, e1f)

    return _tc_final(agg2.reshape(NC, N, D), selfsum2, deg, deg,
                     W_lin, b_lin2)
